# baseline (speedup 1.0000x reference)
import functools

import jax
import jax.numpy as jnp
from jax import lax
from jax.experimental import pallas as pl
from jax.experimental.pallas import tpu as pltpu
from jax.experimental.pallas import tpu_sc as plsc

D_VOCAB = 1_000_000
D_MODEL = 64
N_TOK = 4096 * 50

NC, NS = 2, 16
NW = NC * NS
TOK_PER_W = N_TOK // NW
CHUNK = 128
NCHUNK = TOK_PER_W // CHUNK

K = 256
NG = D_VOCAB // K
TAIL_START = D_VOCAB - K
NCH = 124

_MESH = plsc.VectorSubcoreMesh(core_axis_name="c", subcore_axis_name="s")
_NOTILE = pltpu.CompilerParams(use_tc_tiling_on_sc=False)
_NOLAYOUT = pltpu.CompilerParams(
    use_tc_tiling_on_sc=False, needs_layout_passes=False
)


@functools.partial(
    pl.kernel,
    out_type=jax.ShapeDtypeStruct((D_VOCAB, D_MODEL), jnp.float32),
    mesh=_MESH,
    scratch_types=[
        pltpu.VMEM((D_MODEL, K), jnp.float32),
        pltpu.VMEM((D_MODEL, K), jnp.float32),
        pltpu.VMEM((K, D_MODEL), jnp.float32),
        pltpu.VMEM((K, D_MODEL), jnp.float32),
        pltpu.SemaphoreType.DMA,
        pltpu.SemaphoreType.DMA,
        pltpu.SemaphoreType.DMA,
        pltpu.SemaphoreType.DMA,
    ],
    compiler_params=_NOLAYOUT,
)
def _sc_transpose(w_hbm, wt_hbm, in0, in1, ot0, ot1, is0, is1, os0, os1):
    w = lax.axis_index("s") * NC + lax.axis_index("c")

    def chunk_start(c):
        g = w + NW * c
        tail = jnp.logical_and(w == NW - 1, c == NCH - 1)
        return jnp.where(g < NG, K * g, jnp.where(tail, TAIL_START, 0))

    ins, ots, iss, oss = [in0, in1], [ot0, ot1], [is0, is1], [os0, os1]

    start0 = chunk_start(0)

    @pl.loop(0, D_MODEL)
    def _(d):
        pltpu.async_copy(w_hbm.at[d, pl.ds(start0, K)], in0.at[d], is0)

    @pl.loop(0, NCH, step=2)
    def _(cc):
        for b in range(2):
            c = cc + b

            @pl.when(c + 1 < NCH)
            def _():
                nstart = chunk_start(c + 1)

                @pl.loop(0, D_MODEL)
                def _(d):
                    pltpu.async_copy(
                        w_hbm.at[d, pl.ds(nstart, K)], ins[1 - b].at[d], iss[1 - b]
                    )

            pltpu.make_async_copy(
                w_hbm.at[:, pl.ds(0, K)], ins[b], iss[b]
            ).wait()

            @pl.when(c >= 2)
            def _():
                pltpu.make_async_copy(
                    ots[b], wt_hbm.at[pl.ds(0, K)], oss[b]
                ).wait()

            @pl.loop(0, D_MODEL)
            def _(d):
                col = jnp.full((16,), d, dtype=jnp.int32)
                vals = [ins[b][d, pl.ds(g * 16, 16)] for g in range(K // 16)]
                for g in range(K // 16):
                    row = jnp.arange(16, dtype=jnp.int32) + g * 16
                    plsc.store_scatter(ots[b], [row, col], vals[g])

            pltpu.async_copy(ots[b], wt_hbm.at[pl.ds(chunk_start(c), K)], oss[b])

    for b in range(2):
        pltpu.make_async_copy(ots[b], wt_hbm.at[pl.ds(0, K)], oss[b]).wait()


@functools.partial(
    pl.kernel,
    out_type=jax.ShapeDtypeStruct((N_TOK, D_MODEL), jnp.float32),
    mesh=_MESH,
    scratch_types=[
        pltpu.VMEM((NCHUNK, CHUNK), jnp.int32),
        pltpu.VMEM((CHUNK, D_MODEL), jnp.float32),
        pltpu.VMEM((CHUNK, D_MODEL), jnp.float32),
        pltpu.SemaphoreType.DMA,
        pltpu.SemaphoreType.DMA,
        pltpu.SemaphoreType.DMA,
        pltpu.SemaphoreType.DMA,
    ],
    compiler_params=_NOTILE,
)
def _sc_gather(x_hbm, wt_hbm, out_hbm, idx_v, r0, r1, gs0, gs1, ss0, ss1):
    w = lax.axis_index("s") * NC + lax.axis_index("c")
    pltpu.sync_copy(x_hbm.at[w], idx_v)

    def body(c, carry):
        pltpu.async_copy(wt_hbm.at[idx_v.at[c]], r0, gs0).wait()
        pltpu.sync_copy(r0, out_hbm.at[pl.ds(w * TOK_PER_W + c * CHUNK, CHUNK)])
        return carry

    lax.fori_loop(0, NCHUNK, body, 0)


def kernel(x, W_E):
    w_t = _sc_transpose(W_E)
    x3 = x.reshape(NW, NCHUNK, CHUNK).astype(jnp.int32)
    out = _sc_gather(x3, w_t)
    return out.reshape(4096, 50, D_MODEL)


# speedup vs baseline: 5.0763x; 5.0763x over previous
import functools

import jax
import jax.numpy as jnp
from jax import lax
from jax.experimental import pallas as pl
from jax.experimental.pallas import tpu as pltpu
from jax.experimental.pallas import tpu_sc as plsc

D_VOCAB = 1_000_000
D_MODEL = 64
N_TOK = 4096 * 50
N_ROW = D_VOCAB // 2
D_PACK = D_MODEL * 2

NC, NS = 2, 16
NW = NC * NS
TOK_PER_W = N_TOK // NW
CHUNK = 128
NCHUNK = TOK_PER_W // CHUNK

TP_BLK = 8192

_MESH = plsc.VectorSubcoreMesh(core_axis_name="c", subcore_axis_name="s")
_NOTILE = pltpu.CompilerParams(
    use_tc_tiling_on_sc=False, needs_layout_passes=False
)


def _tp_body(w_ref, o_ref):
    o_ref[...] = w_ref[...].T


def _transpose(w_e):
    grid = pl.cdiv(D_VOCAB, TP_BLK)
    return pl.pallas_call(
        _tp_body,
        grid=(grid,),
        in_specs=[pl.BlockSpec((D_MODEL, TP_BLK), lambda i: (0, i))],
        out_specs=pl.BlockSpec((TP_BLK, D_MODEL), lambda i: (i, 0)),
        out_shape=jax.ShapeDtypeStruct((D_VOCAB, D_MODEL), jnp.float32),
    )(w_e)


@functools.partial(
    pl.kernel,
    out_type=jax.ShapeDtypeStruct((N_TOK, D_MODEL), jnp.float32),
    mesh=_MESH,
    scratch_types=[
        pltpu.VMEM((TOK_PER_W,), jnp.int32),
        pltpu.VMEM((CHUNK,), jnp.int32),
        pltpu.VMEM((CHUNK,), jnp.int32),
        pltpu.VMEM((CHUNK, D_PACK), jnp.float32),
        pltpu.VMEM((CHUNK, D_PACK), jnp.float32),
        pltpu.VMEM((CHUNK, D_MODEL), jnp.float32),
        pltpu.VMEM((CHUNK, D_MODEL), jnp.float32),
        pltpu.SemaphoreType.DMA,
        pltpu.SemaphoreType.DMA,
        pltpu.SemaphoreType.DMA,
        pltpu.SemaphoreType.DMA,
    ],
    compiler_params=_NOTILE,
)
def _sc_gather(x_hbm, wt_hbm, out_hbm, idx_v, p0, p1, r0, r1, c0, c1,
               gs0, gs1, ss0, ss1):
    w = lax.axis_index("s") * NC + lax.axis_index("c")
    ps, rs, cs = [p0, p1], [r0, r1], [c0, c1]
    gss, sss = [gs0, gs1], [ss0, ss1]
    pltpu.sync_copy(x_hbm.at[w], idx_v)

    def fill_rowids(c, pbuf):
        for g in range(CHUNK // 16):
            v = idx_v[pl.ds(c * CHUNK + g * 16, 16)]
            pbuf[pl.ds(g * 16, 16)] = lax.shift_right_logical(v, 1)

    def start_gather(c, b):
        fill_rowids(c, ps[b])
        pltpu.async_copy(wt_hbm.at[ps[b]], rs[b], gss[b])

    def half_select(c, b):
        for g in range(CHUNK // 16):
            v = idx_v[pl.ds(c * CHUNK + g * 16, 16)]
            off = jnp.bitwise_and(v, 1) * D_MODEL
            jrow = jnp.arange(16, dtype=jnp.int32) + g * 16

            @pl.loop(0, D_MODEL)
            def _(col):
                vals = plsc.load_gather(rs[b], [jrow, off + col])
                plsc.store_scatter(
                    cs[b], [jrow, jnp.full((16,), col, jnp.int32)], vals
                )

    start_gather(0, 0)

    @pl.loop(0, NCHUNK, step=2)
    def _(cc):
        for b in range(2):
            c = cc + b

            @pl.when(c >= 1)
            def _():
                pltpu.make_async_copy(
                    cs[1 - b], out_hbm.at[pl.ds(0, CHUNK)], sss[1 - b]
                ).wait()

            @pl.when(c + 1 < NCHUNK)
            def _():
                start_gather(c + 1, 1 - b)

            pltpu.make_async_copy(wt_hbm.at[ps[b]], rs[b], gss[b]).wait()
            half_select(c, b)
            pltpu.async_copy(
                cs[b], out_hbm.at[pl.ds(w * TOK_PER_W + c * CHUNK, CHUNK)], sss[b]
            )

    lastb = (NCHUNK - 1) & 1
    pltpu.make_async_copy(cs[lastb], out_hbm.at[pl.ds(0, CHUNK)], sss[lastb]).wait()


def kernel(x, W_E):
    w_t = _transpose(W_E).reshape(N_ROW, D_PACK)
    x2 = x.reshape(NW, TOK_PER_W).astype(jnp.int32)
    out = _sc_gather(x2, w_t)
    return out.reshape(4096, 50, D_MODEL)


# speedup vs baseline: 7.7433x; 1.5254x over previous
import functools

import jax
import jax.numpy as jnp
from jax import lax
from jax.experimental import pallas as pl
from jax.experimental.pallas import tpu as pltpu
from jax.experimental.pallas import tpu_sc as plsc

D_VOCAB = 1_000_000
D_MODEL = 64
N_TOK = 4096 * 50

NC, NS = 2, 16
NW = NC * NS
TOK_PER_W = N_TOK // NW
CHUNK = 128
NCHUNK = TOK_PER_W // CHUNK

TP_BLK = 8192

_MESH = plsc.VectorSubcoreMesh(core_axis_name="c", subcore_axis_name="s")
_NOTILE = pltpu.CompilerParams(use_tc_tiling_on_sc=False)


def _tp_body(w_ref, o_ref):
    o_ref[...] = w_ref[...].T


def _transpose(w_e):
    grid = pl.cdiv(D_VOCAB, TP_BLK)
    return pl.pallas_call(
        _tp_body,
        grid=(grid,),
        in_specs=[pl.BlockSpec((D_MODEL, TP_BLK), lambda i: (0, i))],
        out_specs=pl.BlockSpec((TP_BLK, D_MODEL), lambda i: (i, 0)),
        out_shape=jax.ShapeDtypeStruct((D_VOCAB, D_MODEL), jnp.float32),
    )(w_e)


@functools.partial(
    pl.kernel,
    out_type=jax.ShapeDtypeStruct((N_TOK, D_MODEL), jnp.float32),
    mesh=_MESH,
    scratch_types=[
        pltpu.VMEM((TOK_PER_W,), jnp.int32),
        pltpu.VMEM((CHUNK, D_MODEL), jnp.float32),
        pltpu.VMEM((CHUNK, D_MODEL), jnp.float32),
        pltpu.SemaphoreType.DMA,
        pltpu.SemaphoreType.DMA,
        pltpu.SemaphoreType.DMA,
        pltpu.SemaphoreType.DMA,
    ],
    compiler_params=_NOTILE,
)
def _sc_gather(x_hbm, wt_hbm, out_hbm, idx_v, r0, r1, gs0, gs1, ss0, ss1):
    w = lax.axis_index("s") * NC + lax.axis_index("c")
    rs, gss, sss = [r0, r1], [gs0, gs1], [ss0, ss1]
    pltpu.sync_copy(x_hbm.at[w], idx_v)

    def start_gather(c, b):
        pltpu.async_copy(
            wt_hbm.at[idx_v.at[pl.ds(c * CHUNK, CHUNK)]], rs[b], gss[b]
        )

    start_gather(0, 0)

    @pl.loop(0, NCHUNK, step=2)
    def _(cc):
        for b in range(2):
            c = cc + b

            @pl.when(c >= 1)
            def _():
                pltpu.make_async_copy(
                    rs[1 - b], out_hbm.at[pl.ds(0, CHUNK)], sss[1 - b]
                ).wait()

            @pl.when(c + 1 < NCHUNK)
            def _():
                start_gather(c + 1, 1 - b)

            pltpu.make_async_copy(
                wt_hbm.at[idx_v.at[pl.ds(0, CHUNK)]], rs[b], gss[b]
            ).wait()

            pltpu.async_copy(
                rs[b], out_hbm.at[pl.ds(w * TOK_PER_W + c * CHUNK, CHUNK)], sss[b]
            )

    lastb = (NCHUNK - 1) & 1
    pltpu.make_async_copy(
        rs[lastb], out_hbm.at[pl.ds(0, CHUNK)], sss[lastb]
    ).wait()


def kernel(x, W_E):
    w_t = _transpose(W_E)
    x2 = x.reshape(NW, TOK_PER_W).astype(jnp.int32)
    out = _sc_gather(x2, w_t)
    return out.reshape(4096, 50, D_MODEL)


# speedup vs baseline: 12.2060x; 1.5763x over previous
import functools

import jax
import jax.numpy as jnp
from jax import lax
from jax.experimental import pallas as pl
from jax.experimental.pallas import tpu as pltpu
from jax.experimental.pallas import tpu_sc as plsc

D_VOCAB = 1_000_000
D_MODEL = 64
N_TOK = 4096 * 50

NC, NS = 2, 16
NW = NC * NS
TOK_PER_W = N_TOK // NW
CHUNK = 128
NCHUNK = TOK_PER_W // CHUNK

TP_BLK = 8192

_MESH = plsc.VectorSubcoreMesh(core_axis_name="c", subcore_axis_name="s")
_NOTILE = pltpu.CompilerParams(use_tc_tiling_on_sc=False)


def _tp_body(w_ref, o_ref):
    t = w_ref[...].T
    o_ref[:, 0:D_MODEL] = t
    o_ref[:, D_MODEL:2 * D_MODEL] = t


def _transpose(w_e):
    grid = pl.cdiv(D_VOCAB, TP_BLK)
    return pl.pallas_call(
        _tp_body,
        grid=(grid,),
        in_specs=[pl.BlockSpec((D_MODEL, TP_BLK), lambda i: (0, i))],
        out_specs=pl.BlockSpec((TP_BLK, 2 * D_MODEL), lambda i: (i, 0)),
        out_shape=jax.ShapeDtypeStruct((D_VOCAB, 2 * D_MODEL), jnp.float32),
    )(w_e)


@functools.partial(
    pl.kernel,
    out_type=jax.ShapeDtypeStruct((N_TOK, D_MODEL), jnp.float32),
    mesh=_MESH,
    scratch_types=[
        pltpu.VMEM((TOK_PER_W,), jnp.int32),
        pltpu.VMEM((CHUNK, 2 * D_MODEL), jnp.float32),
        pltpu.VMEM((CHUNK, 2 * D_MODEL), jnp.float32),
        pltpu.SemaphoreType.DMA,
        pltpu.SemaphoreType.DMA,
        pltpu.SemaphoreType.DMA,
        pltpu.SemaphoreType.DMA,
    ],
    compiler_params=_NOTILE,
)
def _sc_gather(x_hbm, wt_hbm, out_hbm, idx_v, r0, r1, gs0, gs1, ss0, ss1):
    w = lax.axis_index("s") * NC + lax.axis_index("c")
    rs, gss, sss = [r0, r1], [gs0, gs1], [ss0, ss1]
    pltpu.sync_copy(x_hbm.at[w], idx_v)

    def start_gather(c, b):
        pltpu.async_copy(
            wt_hbm.at[idx_v.at[pl.ds(c * CHUNK, CHUNK)]], rs[b], gss[b]
        )

    start_gather(0, 0)

    @pl.loop(0, NCHUNK, step=2)
    def _(cc):
        for b in range(2):
            c = cc + b

            @pl.when(c >= 1)
            def _():
                pltpu.make_async_copy(
                    rs[1 - b].at[:, pl.ds(0, D_MODEL)],
                    out_hbm.at[pl.ds(0, CHUNK)], sss[1 - b]
                ).wait()

            @pl.when(c + 1 < NCHUNK)
            def _():
                start_gather(c + 1, 1 - b)

            pltpu.make_async_copy(
                wt_hbm.at[idx_v.at[pl.ds(0, CHUNK)]], rs[b], gss[b]
            ).wait()

            pltpu.async_copy(
                rs[b].at[:, pl.ds(0, D_MODEL)],
                out_hbm.at[pl.ds(w * TOK_PER_W + c * CHUNK, CHUNK)], sss[b]
            )

    lastb = (NCHUNK - 1) & 1
    pltpu.make_async_copy(
        rs[lastb].at[:, pl.ds(0, D_MODEL)], out_hbm.at[pl.ds(0, CHUNK)], sss[lastb]
    ).wait()


def kernel(x, W_E):
    w_t = _transpose(W_E)
    x2 = x.reshape(NW, TOK_PER_W).astype(jnp.int32)
    out = _sc_gather(x2, w_t)
    return out.reshape(4096, 50, D_MODEL)


# speedup vs baseline: 13.6134x; 1.1153x over previous
import functools

import jax
import jax.numpy as jnp
from jax import lax
from jax.experimental import pallas as pl
from jax.experimental.pallas import tpu as pltpu
from jax.experimental.pallas import tpu_sc as plsc

D_VOCAB = 1_000_000
D_MODEL = 64
N_TOK = 4096 * 50

NC, NS = 2, 16
NW = NC * NS
TOK_PER_W = N_TOK // NW
CHUNK = 128
NCHUNK = TOK_PER_W // CHUNK

TP_BLK = 8192

_MESH = plsc.VectorSubcoreMesh(core_axis_name="c", subcore_axis_name="s")
_NOTILE = pltpu.CompilerParams(use_tc_tiling_on_sc=False)


def _tp_body(w_ref, o_ref):
    o_ref[:, 0:D_MODEL] = w_ref[...].T


def _transpose(w_e):
    grid = pl.cdiv(D_VOCAB, TP_BLK)
    return pl.pallas_call(
        _tp_body,
        grid=(grid,),
        in_specs=[pl.BlockSpec((D_MODEL, TP_BLK), lambda i: (0, i))],
        out_specs=pl.BlockSpec((TP_BLK, 2 * D_MODEL), lambda i: (i, 0)),
        out_shape=jax.ShapeDtypeStruct((D_VOCAB, 2 * D_MODEL), jnp.float32),
    )(w_e)


@functools.partial(
    pl.kernel,
    out_type=jax.ShapeDtypeStruct((N_TOK, D_MODEL), jnp.float32),
    mesh=_MESH,
    scratch_types=[
        pltpu.VMEM((TOK_PER_W,), jnp.int32),
        pltpu.VMEM((CHUNK, 2 * D_MODEL), jnp.float32),
        pltpu.VMEM((CHUNK, 2 * D_MODEL), jnp.float32),
        pltpu.SemaphoreType.DMA,
        pltpu.SemaphoreType.DMA,
        pltpu.SemaphoreType.DMA,
        pltpu.SemaphoreType.DMA,
    ],
    compiler_params=_NOTILE,
)
def _sc_gather(x_hbm, wt_hbm, out_hbm, idx_v, r0, r1, gs0, gs1, ss0, ss1):
    w = lax.axis_index("s") * NC + lax.axis_index("c")
    rs, gss, sss = [r0, r1], [gs0, gs1], [ss0, ss1]
    pltpu.sync_copy(x_hbm.at[w], idx_v)

    def start_gather(c, b):
        pltpu.async_copy(
            wt_hbm.at[idx_v.at[pl.ds(c * CHUNK, CHUNK)]], rs[b], gss[b]
        )

    start_gather(0, 0)

    @pl.loop(0, NCHUNK, step=2)
    def _(cc):
        for b in range(2):
            c = cc + b

            @pl.when(c >= 1)
            def _():
                pltpu.make_async_copy(
                    rs[1 - b].at[:, pl.ds(0, D_MODEL)],
                    out_hbm.at[pl.ds(0, CHUNK)], sss[1 - b]
                ).wait()

            @pl.when(c + 1 < NCHUNK)
            def _():
                start_gather(c + 1, 1 - b)

            pltpu.make_async_copy(
                wt_hbm.at[idx_v.at[pl.ds(0, CHUNK)]], rs[b], gss[b]
            ).wait()

            pltpu.async_copy(
                rs[b].at[:, pl.ds(0, D_MODEL)],
                out_hbm.at[pl.ds(w * TOK_PER_W + c * CHUNK, CHUNK)], sss[b]
            )

    lastb = (NCHUNK - 1) & 1
    pltpu.make_async_copy(
        rs[lastb].at[:, pl.ds(0, D_MODEL)], out_hbm.at[pl.ds(0, CHUNK)], sss[lastb]
    ).wait()


def kernel(x, W_E):
    w_t = _transpose(W_E)
    x2 = x.reshape(NW, TOK_PER_W).astype(jnp.int32)
    out = _sc_gather(x2, w_t)
    return out.reshape(4096, 50, D_MODEL)


# speedup vs baseline: 14.3692x; 1.0555x over previous
import functools

import jax
import jax.numpy as jnp
from jax import lax
from jax.experimental import pallas as pl
from jax.experimental.pallas import tpu as pltpu
from jax.experimental.pallas import tpu_sc as plsc

D_VOCAB = 1_000_000
D_MODEL = 64
N_TOK = 4096 * 50

NC, NS = 2, 16
NW = NC * NS
TOK_PER_W = N_TOK // NW
CHUNK = 128
NCHUNK = TOK_PER_W // CHUNK

TP_BLK = 32768

_MESH = plsc.VectorSubcoreMesh(core_axis_name="c", subcore_axis_name="s")
_NOTILE = pltpu.CompilerParams(use_tc_tiling_on_sc=False)


def _tp_body(w_ref, o_ref):
    o_ref[:, 0:D_MODEL] = w_ref[...].T


def _transpose(w_e):
    grid = pl.cdiv(D_VOCAB, TP_BLK)
    return pl.pallas_call(
        _tp_body,
        grid=(grid,),
        in_specs=[pl.BlockSpec((D_MODEL, TP_BLK), lambda i: (0, i))],
        out_specs=pl.BlockSpec((TP_BLK, 2 * D_MODEL), lambda i: (i, 0)),
        out_shape=jax.ShapeDtypeStruct((D_VOCAB, 2 * D_MODEL), jnp.float32),
    )(w_e)


@functools.partial(
    pl.kernel,
    out_type=jax.ShapeDtypeStruct((N_TOK, D_MODEL), jnp.float32),
    mesh=_MESH,
    scratch_types=[
        pltpu.VMEM((TOK_PER_W,), jnp.int32),
        pltpu.VMEM((CHUNK, 2 * D_MODEL), jnp.float32),
        pltpu.VMEM((CHUNK, 2 * D_MODEL), jnp.float32),
        pltpu.SemaphoreType.DMA,
        pltpu.SemaphoreType.DMA,
        pltpu.SemaphoreType.DMA,
        pltpu.SemaphoreType.DMA,
    ],
    compiler_params=_NOTILE,
)
def _sc_gather(x_hbm, wt_hbm, out_hbm, idx_v, r0, r1, gs0, gs1, ss0, ss1):
    w = lax.axis_index("s") * NC + lax.axis_index("c")
    rs, gss, sss = [r0, r1], [gs0, gs1], [ss0, ss1]
    pltpu.sync_copy(x_hbm.at[w], idx_v)

    def start_gather(c, b):
        pltpu.async_copy(
            wt_hbm.at[idx_v.at[pl.ds(c * CHUNK, CHUNK)]], rs[b], gss[b]
        )

    start_gather(0, 0)

    @pl.loop(0, NCHUNK, step=2)
    def _(cc):
        for b in range(2):
            c = cc + b

            @pl.when(c >= 1)
            def _():
                pltpu.make_async_copy(
                    rs[1 - b].at[:, pl.ds(0, D_MODEL)],
                    out_hbm.at[pl.ds(0, CHUNK)], sss[1 - b]
                ).wait()

            @pl.when(c + 1 < NCHUNK)
            def _():
                start_gather(c + 1, 1 - b)

            pltpu.make_async_copy(
                wt_hbm.at[idx_v.at[pl.ds(0, CHUNK)]], rs[b], gss[b]
            ).wait()

            pltpu.async_copy(
                rs[b].at[:, pl.ds(0, D_MODEL)],
                out_hbm.at[pl.ds(w * TOK_PER_W + c * CHUNK, CHUNK)], sss[b]
            )

    lastb = (NCHUNK - 1) & 1
    pltpu.make_async_copy(
        rs[lastb].at[:, pl.ds(0, D_MODEL)], out_hbm.at[pl.ds(0, CHUNK)], sss[lastb]
    ).wait()


def kernel(x, W_E):
    w_t = _transpose(W_E)
    x2 = x.reshape(NW, TOK_PER_W).astype(jnp.int32)
    out = _sc_gather(x2, w_t)
    return out.reshape(4096, 50, D_MODEL)
